# baseline (speedup 1.0000x reference)
import functools

import jax
import jax.numpy as jnp
from jax import lax
from jax.experimental import pallas as pl
from jax.experimental.pallas import tpu as pltpu
from jax.experimental.pallas import tpu_sc as plsc

NUM_EMB = 8192
DIM = 64
ROWS = 16384
COMMITMENT_COST = 0.25

BLK_R = 256
BLK_K = 2048
N_R = ROWS // BLK_R
N_KC = NUM_EMB // BLK_K


def _dist_body(x_ref, rn_ref, w_ref, wn_ref, idx_ref, md_ref):
    x = x_ref[...]
    rn = rn_ref[...]
    best_d = jnp.full((BLK_R, 1), jnp.inf, jnp.float32)
    best_i = jnp.zeros((BLK_R, 1), jnp.int32)
    for c in range(N_KC):
        w = w_ref[pl.ds(c * BLK_K, BLK_K), :]
        wn = wn_ref[:, pl.ds(c * BLK_K, BLK_K)]
        mm = lax.dot_general(x, w, (((1,), (1,)), ((), ())),
                             preferred_element_type=jnp.float32)
        d2 = (rn + wn) - 2.0 * mm
        dist = jnp.sqrt(jnp.maximum(d2, 0.0))
        m = jnp.min(dist, axis=1, keepdims=True)
        iota = lax.broadcasted_iota(jnp.int32, (BLK_R, BLK_K), 1) + (c * BLK_K)
        im = jnp.min(jnp.where(dist == m, iota, NUM_EMB),
                     axis=1, keepdims=True)
        upd = m < best_d
        best_i = jnp.where(upd, im, best_i)
        best_d = jnp.where(upd, m, best_d)
    idx_ref[...] = best_i
    md_ref[...] = best_d * best_d


def _assign_codes(flat, rn, weight, wn, interpret=False):
    return pl.pallas_call(
        _dist_body,
        grid=(N_R,),
        in_specs=[
            pl.BlockSpec((BLK_R, DIM), lambda i: (i, 0)),
            pl.BlockSpec((BLK_R, 1), lambda i: (i, 0)),
            pl.BlockSpec((NUM_EMB, DIM), lambda i: (0, 0)),
            pl.BlockSpec((1, NUM_EMB), lambda i: (0, 0)),
        ],
        out_specs=[
            pl.BlockSpec((BLK_R, 1), lambda i: (i, 0)),
            pl.BlockSpec((BLK_R, 1), lambda i: (i, 0)),
        ],
        out_shape=[
            jax.ShapeDtypeStruct((ROWS, 1), jnp.int32),
            jax.ShapeDtypeStruct((ROWS, 1), jnp.float32),
        ],
        interpret=interpret,
    )(flat, rn, weight, wn)


_NW = 32
_B_PER_W = ROWS // _NW
_IDX_CH = 128
_CH_PER_W = _B_PER_W // _IDX_CH
_GDIM = 128


def _make_sc_gather():
    mesh = plsc.VectorSubcoreMesh(core_axis_name="c", subcore_axis_name="s")

    @functools.partial(
        pl.kernel,
        mesh=mesh,
        out_type=jax.ShapeDtypeStruct((ROWS, _GDIM), jnp.float32),
        scratch_types=[
            pltpu.VMEM((_CH_PER_W, _IDX_CH), jnp.int32),
            pltpu.VMEM((_B_PER_W, _GDIM), jnp.float32),
            pltpu.SemaphoreType.DMA,
        ],
    )
    def gather_k(table_hbm, idx_hbm, out_hbm, idx_v, rows_v, sem):
        wid = lax.axis_index("s") * 2 + lax.axis_index("c")
        base = wid * _B_PER_W
        pltpu.sync_copy(idx_hbm.at[pl.ds(wid * _CH_PER_W, _CH_PER_W)], idx_v)
        copies = []
        for j in range(_CH_PER_W):
            copies.append(pltpu.async_copy(
                table_hbm.at[idx_v.at[j]],
                rows_v.at[pl.ds(j * _IDX_CH, _IDX_CH)], sem))
        for cp in copies:
            cp.wait()
        pltpu.sync_copy(rows_v, out_hbm.at[pl.ds(base, _B_PER_W)])

    return gather_k


_sc_gather_cache = []


def _sc_gather(table, idx2d):
    if not _sc_gather_cache:
        _sc_gather_cache.append(_make_sc_gather())
    return _sc_gather_cache[0](table, idx2d)


def kernel(inputs, weight):
    x = jnp.transpose(inputs, (0, 2, 3, 1))
    input_shape = x.shape
    flat = x.reshape(-1, DIM)
    rn = jnp.sum(flat * flat, axis=1, keepdims=True)
    wn = jnp.sum(weight * weight, axis=1)[None, :]
    idx, md = _assign_codes(flat, rn, weight, wn)
    idx2d = idx.reshape(ROWS // _IDX_CH, _IDX_CH)
    wpad = jnp.pad(weight, ((0, 0), (0, _GDIM - DIM)))
    quantized = _sc_gather(wpad, idx2d)[:, :DIM]
    m = jnp.sum(md) / (ROWS * DIM)
    c_loss = m + COMMITMENT_COST * m
    quantized = quantized.reshape(input_shape)
    quantized = jnp.transpose(quantized, (0, 3, 1, 2))
    return (c_loss, quantized)
